# baseline (speedup 1.0000x reference)
import functools

import jax
import jax.numpy as jnp
from jax import lax
from jax.experimental import pallas as pl
from jax.experimental.pallas import tpu as pltpu
from jax.experimental.pallas import tpu_sc as plsc

_BATCH = 16384
_DIM = 64
_NC = 2
_NS = 16
_NW = _NC * _NS
_BPW = _BATCH // _NW
_C = 128
_NCHUNK = _BPW // _C
_L = 16
_EPS = 1e-9
_MARGIN = 1.0


def _build_sc_call():
    f32 = jnp.float32
    i32 = jnp.int32
    mesh = plsc.VectorSubcoreMesh(core_axis_name="c", subcore_axis_name="s")
    scratch = (
        [pltpu.VMEM((_C,), i32) for _ in range(6)]
        + [pltpu.VMEM((_C, _DIM), f32) for _ in range(12)]
        + [pltpu.VMEM((_L,), f32), pltpu.SemaphoreType.DMA]
    )

    @functools.partial(
        pl.kernel,
        mesh=mesh,
        out_type=jax.ShapeDtypeStruct((_NW, _L), f32),
        scratch_types=scratch,
        compiler_params=pltpu.CompilerParams(
            needs_layout_passes=False, use_tc_tiling_on_sc=False),
    )
    def sc_fn(ph, pr, pt, nh, nr, nt,
              ent_emb, ent_cov, rel_emb, rel_cov,
              out,
              phic, pric, ptic, nhic, nric, ntic,
              phm, phv, prm, prv, ptm, ptv,
              nhm, nhv, nrm, nrv, ntm, ntv,
              totv, sem):
        wid = lax.axis_index("s") * _NC + lax.axis_index("c")
        base = wid * _BPW
        iota = lax.iota(i32, _L)
        total = jnp.zeros((_L,), f32)
        for g in range(_NCHUNK):
            off = base + g * _C
            pltpu.sync_copy(ph.at[pl.ds(off, _C)], phic)
            pltpu.sync_copy(pr.at[pl.ds(off, _C)], pric)
            pltpu.sync_copy(pt.at[pl.ds(off, _C)], ptic)
            pltpu.sync_copy(nh.at[pl.ds(off, _C)], nhic)
            pltpu.sync_copy(nr.at[pl.ds(off, _C)], nric)
            pltpu.sync_copy(nt.at[pl.ds(off, _C)], ntic)
            cps = [
                pltpu.async_copy(ent_emb.at[phic], phm, sem),
                pltpu.async_copy(ent_cov.at[phic], phv, sem),
                pltpu.async_copy(rel_emb.at[pric], prm, sem),
                pltpu.async_copy(rel_cov.at[pric], prv, sem),
                pltpu.async_copy(ent_emb.at[ptic], ptm, sem),
                pltpu.async_copy(ent_cov.at[ptic], ptv, sem),
                pltpu.async_copy(ent_emb.at[nhic], nhm, sem),
                pltpu.async_copy(ent_cov.at[nhic], nhv, sem),
                pltpu.async_copy(rel_emb.at[nric], nrm, sem),
                pltpu.async_copy(rel_cov.at[nric], nrv, sem),
                pltpu.async_copy(ent_emb.at[ntic], ntm, sem),
                pltpu.async_copy(ent_cov.at[ntic], ntv, sem),
            ]
            for cp in cps:
                cp.wait()
            for rg in range(_C // _L):
                rows = rg * _L + iota

                def dim_body(d, carry, rows=rows):
                    accp, accn = carry
                    col = jnp.full((_L,), d, i32)
                    idx = [rows, col]
                    hm = plsc.load_gather(phm, idx)
                    hv = plsc.load_gather(phv, idx)
                    rm = plsc.load_gather(prm, idx)
                    rv = plsc.load_gather(prv, idx)
                    tm = plsc.load_gather(ptm, idx)
                    tv = plsc.load_gather(ptv, idx)
                    ev = tv + hv
                    diff = rm - (tm - hm)
                    d2 = diff * diff
                    accp = accp + (ev + d2) / (rv + _EPS) + (rv + d2) / (ev + _EPS)
                    hm = plsc.load_gather(nhm, idx)
                    hv = plsc.load_gather(nhv, idx)
                    rm = plsc.load_gather(nrm, idx)
                    rv = plsc.load_gather(nrv, idx)
                    tm = plsc.load_gather(ntm, idx)
                    tv = plsc.load_gather(ntv, idx)
                    ev = tv + hv
                    diff = rm - (tm - hm)
                    d2 = diff * diff
                    accn = accn + (ev + d2) / (rv + _EPS) + (rv + d2) / (ev + _EPS)
                    return accp, accn

                accp, accn = lax.fori_loop(
                    0, _DIM, dim_body,
                    (jnp.zeros((_L,), f32), jnp.zeros((_L,), f32)))
                total = total + jnp.maximum((accp - accn) * 0.25 + _MARGIN, 0.0)
        totv[...] = total
        pltpu.sync_copy(totv, out.at[wid])

    return sc_fn


_sc_call = _build_sc_call()


def kernel(pos, neg, ent_emb, ent_cov, rel_emb, rel_cov):
    ph, pr, pt = pos[:, 0], pos[:, 1], pos[:, 2]
    nh, nr, nt = neg[:, 0], neg[:, 1], neg[:, 2]
    parts = _sc_call(ph, pr, pt, nh, nr, nt, ent_emb, ent_cov, rel_emb, rel_cov)
    return jnp.sum(parts) / jnp.float32(_BATCH)


# speedup vs baseline: 2.0052x; 2.0052x over previous
import functools

import jax
import jax.numpy as jnp
from jax import lax
from jax.experimental import pallas as pl
from jax.experimental.pallas import tpu as pltpu
from jax.experimental.pallas import tpu_sc as plsc

_E = 1000000
_D = 64
_B = 16384
_NWORK = 32
_NCOL = 7813
_W = 384
_NWIN = 82
_CLAMP = 999552
_TAIL0 = 999936
_ECAP = 5120
_RCAP = 2560
_WCAP = 1024
_CHUNK = 1024
_DUMP = 6 * _B
_OUTR = _DUMP + 16 * _NWORK
_EPS = 1e-9
_MARGIN = 1.0

_i32 = jnp.int32
_f32 = jnp.float32


def _build_sweep():
    mesh = plsc.VectorSubcoreMesh(core_axis_name="c", subcore_axis_name="s")
    scratch = (
        [pltpu.VMEM((_CHUNK,), _i32)]
        + [pltpu.VMEM((_ECAP,), _i32) for _ in range(2)]
        + [pltpu.VMEM((_RCAP,), _i32) for _ in range(2)]
        + [pltpu.VMEM((_WCAP,), _i32) for _ in range(2)]
        + [pltpu.VMEM((8, _W), _f32) for _ in range(16)]
        + [pltpu.VMEM((_NTAIL := 64, 64), _f32) for _ in range(4)]
        + [pltpu.VMEM((16, 128), _f32), pltpu.VMEM((16,), _i32),
           pltpu.SemaphoreType.DMA]
    )

    @functools.partial(
        pl.kernel,
        mesh=mesh,
        out_type=jax.ShapeDtypeStruct((_OUTR, 128), _f32),
        scratch_types=scratch,
        compiler_params=pltpu.CompilerParams(
            needs_layout_passes=False, use_tc_tiling_on_sc=True),
    )
    def sweep(ph, pr, pt, nh, nr, nt,
              tte, ttc, ttr, ttv,
              tle, tlc, tlr, tlv,
              out,
              chunkbuf, ei, es, ri, rs, wi, ws,
              e0, e1, e2, e3, e4, e5, e6, e7,
              c0, c1, c2, c3, c4, c5, c6, c7,
              tb0, tb1, tb2, tb3,
              stage, slotv, sem):
        eslab = [e0, e1, e2, e3, e4, e5, e6, e7]
        cslab = [c0, c1, c2, c3, c4, c5, c6, c7]
        wid = lax.axis_index("s") * 2 + lax.axis_index("c")
        lo_col = (_NCOL * wid) // _NWORK
        hi_col = (_NCOL * (wid + 1)) // _NWORK
        lo = lo_col * 128
        hi = jnp.minimum(hi_col * 128, _E)
        iota = lax.iota(_i32, 16)
        rowc = [jnp.full((16,), r, _i32) for r in range(8)]

        def scan_list(srcs_roles, ilist, slist):
            cnt = jnp.int32(0)
            for src, role in srcs_roles:
                for c in range(_B // _CHUNK):
                    pltpu.sync_copy(src.at[pl.ds(c * _CHUNK, _CHUNK)],
                                    chunkbuf)
                    base_slot = role * _B + c * _CHUNK

                    def sbody(v, cnt, base_slot=base_slot):
                        x16 = chunkbuf[pl.ds(v * 16, 16)]
                        m = (x16 >= lo) & (x16 < hi)
                        s16 = base_slot + v * 16 + iota
                        plsc.store_compressed(ilist.at[pl.ds(cnt, 16)], x16,
                                              mask=m)
                        plsc.store_compressed(slist.at[pl.ds(cnt, 16)], s16,
                                              mask=m)
                        return cnt + plsc.all_reduce_population_count(m)[0]

                    cnt = lax.fori_loop(0, _CHUNK // 16, sbody, cnt)
            return cnt

        cnt_e = scan_list([(ph, 0), (pt, 2), (nh, 3), (nt, 5)], ei, es)
        cnt_r = scan_list([(pr, 1), (nr, 4)], ri, rs)

        def filter_window(ilist, slist, cnt, i0, width):
            def fbody(v, wc):
                x16 = ilist[pl.ds(v * 16, 16)]
                s16 = slist[pl.ds(v * 16, 16)]
                valid = (v * 16 + iota) < cnt
                m = valid & (x16 >= i0) & (x16 < i0 + width)
                plsc.store_compressed(wi.at[pl.ds(wc, 16)], x16 - i0, mask=m)
                plsc.store_compressed(ws.at[pl.ds(wc, 16)], s16, mask=m)
                return wc + plsc.all_reduce_population_count(m)[0]

            return lax.fori_loop(0, (cnt + 15) // 16, fbody, jnp.int32(0))

        def extract_groups(wcnt, gather_pair):
            def gbody(g, _):
                c16 = wi[pl.ds(g * 16, 16)]
                s16 = ws[pl.ds(g * 16, 16)]
                valid = (g * 16 + iota) < wcnt
                c16 = jnp.where(valid, c16, 0)
                slot16 = jnp.where(valid, s16, _DUMP + wid * 16 + iota)
                slotv[...] = slot16
                for d in range(_D):
                    ve, vc = gather_pair(d, c16)
                    plsc.store_scatter(stage, [iota, jnp.full((16,), d, _i32)],
                                       ve)
                    plsc.store_scatter(stage,
                                       [iota, jnp.full((16,), 64 + d, _i32)],
                                       vc)
                pltpu.async_copy(stage, out.at[slotv], sem).wait()
                return 0

            lax.fori_loop(0, (wcnt + 15) // 16, gbody, 0)

        def wbody(k, _):
            i0 = jnp.minimum(lo + k * _W, _CLAMP)
            wcnt = filter_window(ei, es, cnt_e, i0, _W)
            cps = [pltpu.async_copy(tte.at[pl.ds(dc * 8, 8), pl.ds(i0, _W)],
                                    eslab[dc], sem) for dc in range(8)]
            cps += [pltpu.async_copy(ttc.at[pl.ds(dc * 8, 8), pl.ds(i0, _W)],
                                     cslab[dc], sem) for dc in range(8)]
            for cp in cps:
                cp.wait()

            def epair(d, c16):
                idx = [rowc[d % 8], c16]
                return (plsc.load_gather(eslab[d // 8], idx),
                        plsc.load_gather(cslab[d // 8], idx))

            extract_groups(wcnt, epair)
            wcnt = filter_window(ri, rs, cnt_r, i0, _W)
            cps = [pltpu.async_copy(ttr.at[pl.ds(dc * 8, 8), pl.ds(i0, _W)],
                                    eslab[dc], sem) for dc in range(8)]
            cps += [pltpu.async_copy(ttv.at[pl.ds(dc * 8, 8), pl.ds(i0, _W)],
                                     cslab[dc], sem) for dc in range(8)]
            for cp in cps:
                cp.wait()
            extract_groups(wcnt, epair)
            return 0

        lax.fori_loop(0, _NWIN, wbody, 0)

        @pl.when(hi == _E)
        def _tail():
            pltpu.sync_copy(tle, tb0)
            pltpu.sync_copy(tlc, tb1)
            pltpu.sync_copy(tlr, tb2)
            pltpu.sync_copy(tlv, tb3)

            def tpair_ent(d, c16):
                idx = [c16, jnp.full((16,), d, _i32)]
                return plsc.load_gather(tb0, idx), plsc.load_gather(tb1, idx)

            def tpair_rel(d, c16):
                idx = [c16, jnp.full((16,), d, _i32)]
                return plsc.load_gather(tb2, idx), plsc.load_gather(tb3, idx)

            wcnt = filter_window(ei, es, cnt_e, jnp.int32(_TAIL0), 64)
            extract_groups(wcnt, tpair_ent)
            wcnt = filter_window(ri, rs, cnt_r, jnp.int32(_TAIL0), 64)
            extract_groups(wcnt, tpair_rel)

    return sweep


def _build_score():
    mesh = plsc.VectorSubcoreMesh(core_axis_name="c", subcore_axis_name="s")
    _C = 128
    scratch = ([pltpu.VMEM((_C, 128), _f32) for _ in range(6)]
               + [pltpu.VMEM((128,), _f32), pltpu.SemaphoreType.DMA])

    @functools.partial(
        pl.kernel,
        mesh=mesh,
        out_type=jax.ShapeDtypeStruct((_NWORK, 128), _f32),
        scratch_types=scratch,
        compiler_params=pltpu.CompilerParams(
            needs_layout_passes=False, use_tc_tiling_on_sc=True),
    )
    def score(rows, out, bh, br, bt, bnh, bnr, bnt, totv, sem):
        wid = lax.axis_index("s") * 2 + lax.axis_index("c")
        b0 = wid * (_B // _NWORK)
        iota = lax.iota(_i32, 16)
        tot = jnp.float32(0.0)
        for c in range(_B // _NWORK // _C):
            base = b0 + c * _C
            bufs = [bh, br, bt, bnh, bnr, bnt]
            cps = [pltpu.async_copy(rows.at[pl.ds(role * _B + base, _C)],
                                    bufs[role], sem) for role in range(6)]
            for cp in cps:
                cp.wait()

            def rbody(r, tot):
                accp = jnp.zeros((16,), _f32)
                accn = jnp.zeros((16,), _f32)
                for l in range(4):
                    sl = pl.ds(l * 16, 16)
                    sc = pl.ds(64 + l * 16, 16)
                    hm, hv = bh[r, sl], bh[r, sc]
                    rm, rv = br[r, sl], br[r, sc]
                    tm, tv = bt[r, sl], bt[r, sc]
                    ev = tv + hv
                    diff = rm - (tm - hm)
                    d2 = diff * diff
                    accp = accp + (ev + d2) / (rv + _EPS) + (rv + d2) / (ev + _EPS)
                    hm, hv = bnh[r, sl], bnh[r, sc]
                    rm, rv = bnr[r, sl], bnr[r, sc]
                    tm, tv = bnt[r, sl], bnt[r, sc]
                    ev = tv + hv
                    diff = rm - (tm - hm)
                    d2 = diff * diff
                    accn = accn + (ev + d2) / (rv + _EPS) + (rv + d2) / (ev + _EPS)
                s = jnp.sum(accp - accn)
                return tot + jnp.maximum(s * 0.25 + _MARGIN, 0.0)

            tot = lax.fori_loop(0, _C, rbody, tot)
        for l in range(8):
            totv[pl.ds(l * 16, 16)] = jnp.where(
                (iota == 0) & (l == 0), tot, 0.0)
        pltpu.sync_copy(totv, out.at[wid])

    return score


_sweep_call = _build_sweep()
_score_call = _build_score()


def kernel(pos, neg, ent_emb, ent_cov, rel_emb, rel_cov):
    ph, pr, pt = pos[:, 0], pos[:, 1], pos[:, 2]
    nh, nr, nt = neg[:, 0], neg[:, 1], neg[:, 2]
    rows = _sweep_call(
        ph, pr, pt, nh, nr, nt,
        ent_emb.T, ent_cov.T, rel_emb.T, rel_cov.T,
        ent_emb[_TAIL0:], ent_cov[_TAIL0:],
        rel_emb[_TAIL0:], rel_cov[_TAIL0:])
    parts = _score_call(rows)
    return jnp.sum(parts) / jnp.float32(_B)


# speedup vs baseline: 2.7619x; 1.3774x over previous
import functools

import jax
import jax.numpy as jnp
from jax import lax
from jax.experimental import pallas as pl
from jax.experimental.pallas import tpu as pltpu
from jax.experimental.pallas import tpu_sc as plsc

_E = 1000000
_D = 64
_B = 16384
_NWORK = 32
_NCOL = 7813
_W = 384
_NWIN = 82
_CLAMP = 999552
_TAIL0 = 999936
_ECAP = 3584
_RCAP = 1792
_WCAP = 512
_CHUNK = 1024
_DUMP = 6 * _B
_OUTR = _DUMP + 16 * _NWORK
_EPS = 1e-9
_MARGIN = 1.0

_i32 = jnp.int32
_f32 = jnp.float32


def _build_sweep():
    mesh = plsc.VectorSubcoreMesh(core_axis_name="c", subcore_axis_name="s")
    scratch = (
        [pltpu.VMEM((_CHUNK,), _i32)]
        + [pltpu.VMEM((_ECAP,), _i32) for _ in range(2)]
        + [pltpu.VMEM((_RCAP,), _i32) for _ in range(2)]
        + [pltpu.VMEM((_WCAP,), _i32) for _ in range(2)]
        + [pltpu.VMEM((_D, _W), _f32) for _ in range(4)]
        + [pltpu.VMEM((_D, 64), _f32) for _ in range(2)]
        + [pltpu.VMEM((16, 128), _f32), pltpu.VMEM((16,), _i32),
           pltpu.SemaphoreType.DMA, pltpu.SemaphoreType.DMA]
    )

    @functools.partial(
        pl.kernel,
        mesh=mesh,
        out_type=jax.ShapeDtypeStruct((_OUTR, 128), _f32),
        scratch_types=scratch,
        compiler_params=pltpu.CompilerParams(
            needs_layout_passes=False, use_tc_tiling_on_sc=True),
    )
    def sweep(ph, pr, pt, nh, nr, nt,
              tte, ttc, ttr, ttv,
              tle, tlc, tlr, tlv,
              out,
              chunkbuf, ei, es, ri, rs, wi, ws,
              se, sc, sre, src_, tb0, tb1,
              stage, slotv, sem, scat):
        wid = lax.axis_index("s") * 2 + lax.axis_index("c")
        lo_col = (_NCOL * wid) // _NWORK
        hi_col = (_NCOL * (wid + 1)) // _NWORK
        lo = lo_col * 128
        hi = jnp.minimum(hi_col * 128, _E)
        iota = lax.iota(_i32, 16)

        def scan_list(srcs_roles, ilist, slist):
            cnt = jnp.int32(0)
            for src, role in srcs_roles:
                for c in range(_B // _CHUNK):
                    pltpu.sync_copy(src.at[pl.ds(c * _CHUNK, _CHUNK)],
                                    chunkbuf)
                    base_slot = role * _B + c * _CHUNK

                    def sbody(v, cnt, base_slot=base_slot):
                        x16 = chunkbuf[pl.ds(v * 16, 16)]
                        m = (x16 >= lo) & (x16 < hi)
                        s16 = base_slot + v * 16 + iota
                        plsc.store_compressed(ilist.at[pl.ds(cnt, 16)], x16,
                                              mask=m)
                        plsc.store_compressed(slist.at[pl.ds(cnt, 16)], s16,
                                              mask=m)
                        return cnt + plsc.all_reduce_population_count(m)[0]

                    cnt = lax.fori_loop(0, _CHUNK // 16, sbody, cnt)
            return cnt

        cnt_e = scan_list([(ph, 0), (pt, 2), (nh, 3), (nt, 5)], ei, es)
        cnt_r = scan_list([(pr, 1), (nr, 4)], ri, rs)

        def filter_window(ilist, slist, cnt, i0, width):
            def fbody(v, wc):
                x16 = ilist[pl.ds(v * 16, 16)]
                s16 = slist[pl.ds(v * 16, 16)]
                valid = (v * 16 + iota) < cnt
                m = valid & (x16 >= i0) & (x16 < i0 + width)
                plsc.store_compressed(wi.at[pl.ds(wc, 16)], x16 - i0, mask=m)
                plsc.store_compressed(ws.at[pl.ds(wc, 16)], s16, mask=m)
                return wc + plsc.all_reduce_population_count(m)[0]

            return lax.fori_loop(0, (cnt + 15) // 16, fbody, jnp.int32(0))

        def extract_groups(wcnt, buf_e, buf_c):
            def gbody(g, _):
                c16 = wi[pl.ds(g * 16, 16)]
                s16 = ws[pl.ds(g * 16, 16)]
                valid = (g * 16 + iota) < wcnt
                c16 = jnp.where(valid, c16, 0)
                slot16 = jnp.where(valid, s16, _DUMP + wid * 16 + iota)
                slotv[...] = slot16
                for d in range(_D):
                    rd = jnp.full((16,), d, _i32)
                    ve = plsc.load_gather(buf_e, [rd, c16])
                    vc = plsc.load_gather(buf_c, [rd, c16])
                    plsc.store_scatter(stage, [iota, rd], ve)
                    plsc.store_scatter(stage, [iota, rd + 64], vc)
                pltpu.async_copy(stage, out.at[slotv], scat).wait()
                return 0

            lax.fori_loop(0, (wcnt + 15) // 16, gbody, 0)

        def fire(tbl_e, tbl_c, i0, be, bc):
            pltpu.async_copy(tbl_e.at[:, pl.ds(i0, _W)], be, sem)
            pltpu.async_copy(tbl_c.at[:, pl.ds(i0, _W)], bc, sem)

        def drain(tbl_e, tbl_c, be, bc):
            pltpu.make_async_copy(tbl_e.at[:, pl.ds(0, _W)], be, sem).wait()
            pltpu.make_async_copy(tbl_c.at[:, pl.ds(0, _W)], bc, sem).wait()

        fire(tte, ttc, jnp.minimum(lo, _CLAMP), se, sc)

        def wbody(k, _):
            i0 = jnp.minimum(lo + k * _W, _CLAMP)
            wcnt = filter_window(ei, es, cnt_e, i0, _W)
            drain(tte, ttc, se, sc)
            fire(ttr, ttv, i0, sre, src_)
            extract_groups(wcnt, se, sc)
            wcnt = filter_window(ri, rs, cnt_r, i0, _W)
            drain(ttr, ttv, sre, src_)
            i0n = jnp.minimum(lo + (k + 1) * _W, _CLAMP)
            fire(tte, ttc, i0n, se, sc)
            extract_groups(wcnt, sre, src_)
            return 0

        lax.fori_loop(0, _NWIN, wbody, 0)
        drain(tte, ttc, se, sc)

        @pl.when(hi == _E)
        def _tail():
            pltpu.sync_copy(tle, tb0)
            pltpu.sync_copy(tlc, tb1)
            wcnt = filter_window(ei, es, cnt_e, jnp.int32(_TAIL0), 64)
            extract_groups(wcnt, tb0, tb1)
            pltpu.sync_copy(tlr, tb0)
            pltpu.sync_copy(tlv, tb1)
            wcnt = filter_window(ri, rs, cnt_r, jnp.int32(_TAIL0), 64)
            extract_groups(wcnt, tb0, tb1)

    return sweep


def _build_score():
    mesh = plsc.VectorSubcoreMesh(core_axis_name="c", subcore_axis_name="s")
    _C = 128
    scratch = ([pltpu.VMEM((_C, 128), _f32) for _ in range(6)]
               + [pltpu.VMEM((128,), _f32), pltpu.SemaphoreType.DMA])

    @functools.partial(
        pl.kernel,
        mesh=mesh,
        out_type=jax.ShapeDtypeStruct((_NWORK, 128), _f32),
        scratch_types=scratch,
        compiler_params=pltpu.CompilerParams(
            needs_layout_passes=False, use_tc_tiling_on_sc=True),
    )
    def score(rows, out, bh, br, bt, bnh, bnr, bnt, totv, sem):
        wid = lax.axis_index("s") * 2 + lax.axis_index("c")
        b0 = wid * (_B // _NWORK)
        iota = lax.iota(_i32, 16)
        tot = jnp.float32(0.0)
        for c in range(_B // _NWORK // _C):
            base = b0 + c * _C
            bufs = [bh, br, bt, bnh, bnr, bnt]
            cps = [pltpu.async_copy(rows.at[pl.ds(role * _B + base, _C)],
                                    bufs[role], sem) for role in range(6)]
            for cp in cps:
                cp.wait()

            def rbody(r, tot):
                accp = jnp.zeros((16,), _f32)
                accn = jnp.zeros((16,), _f32)
                for l in range(4):
                    sl = pl.ds(l * 16, 16)
                    sc = pl.ds(64 + l * 16, 16)
                    hm, hv = bh[r, sl], bh[r, sc]
                    rm, rv = br[r, sl], br[r, sc]
                    tm, tv = bt[r, sl], bt[r, sc]
                    ev = tv + hv
                    diff = rm - (tm - hm)
                    d2 = diff * diff
                    accp = accp + (ev + d2) / (rv + _EPS) + (rv + d2) / (ev + _EPS)
                    hm, hv = bnh[r, sl], bnh[r, sc]
                    rm, rv = bnr[r, sl], bnr[r, sc]
                    tm, tv = bnt[r, sl], bnt[r, sc]
                    ev = tv + hv
                    diff = rm - (tm - hm)
                    d2 = diff * diff
                    accn = accn + (ev + d2) / (rv + _EPS) + (rv + d2) / (ev + _EPS)
                s = jnp.sum(accp - accn)
                return tot + jnp.maximum(s * 0.25 + _MARGIN, 0.0)

            tot = lax.fori_loop(0, _C, rbody, tot)
        for l in range(8):
            totv[pl.ds(l * 16, 16)] = jnp.where(
                (iota == 0) & (l == 0), tot, 0.0)
        pltpu.sync_copy(totv, out.at[wid])

    return score


_sweep_call = _build_sweep()
_score_call = _build_score()


def kernel(pos, neg, ent_emb, ent_cov, rel_emb, rel_cov):
    ph, pr, pt = pos[:, 0], pos[:, 1], pos[:, 2]
    nh, nr, nt = neg[:, 0], neg[:, 1], neg[:, 2]
    rows = _sweep_call(
        ph, pr, pt, nh, nr, nt,
        ent_emb.T, ent_cov.T, rel_emb.T, rel_cov.T,
        ent_emb[_TAIL0:].T, ent_cov[_TAIL0:].T,
        rel_emb[_TAIL0:].T, rel_cov[_TAIL0:].T)
    parts = _score_call(rows)
    return jnp.sum(parts) / jnp.float32(_B)
